# baseline (speedup 1.0000x reference)
import functools

import jax
import jax.numpy as jnp
from jax import lax
from jax.experimental import pallas as pl
from jax.experimental.pallas import tpu as pltpu
from jax.experimental.pallas import tpu_sc as plsc

HIDDEN = 64
CHUNK = 128
NBUF = 4


@functools.lru_cache(maxsize=None)
def _make_gather(B, DS, NB):
    info = plsc.get_sparse_core_info()
    nc, ns = info.num_cores, info.num_subcores
    nw = nc * ns
    rows_per_w = B // nw
    ng = rows_per_w // CHUNK
    npg = NB // CHUNK
    assert rows_per_w % CHUNK == 0 and ng % NBUF == 0 and NB % CHUNK == 0

    mesh = plsc.VectorSubcoreMesh(core_axis_name="c", subcore_axis_name="s")

    @functools.partial(
        pl.kernel,
        mesh=mesh,
        compiler_params=pltpu.CompilerParams(needs_layout_passes=False),
        out_type=jax.ShapeDtypeStruct((DS, HIDDEN, NB), jnp.float32),
        scratch_types=(
            [pltpu.VMEM((rows_per_w,), jnp.int32)]
            + [pltpu.VMEM((rows_per_w,), jnp.int32)]
            + [pltpu.VMEM((CHUNK, 2 * HIDDEN), jnp.float32) for _ in range(NBUF)]
            + [pltpu.VMEM((HIDDEN, CHUNK), jnp.float32) for _ in range(2)]
            + [pltpu.SemaphoreType.DMA for _ in range(NBUF)]
        ),
    )
    def gather_kernel(idx_hbm, pairs_hbm, out_hbm, idx_v, pidx_v, *rest):
        bufs = rest[:NBUF]
        tbufs = rest[NBUF:NBUF + 2]
        sems = rest[NBUF + 2:]
        wid = lax.axis_index("s") * nc + lax.axis_index("c")
        base = wid * rows_per_w

        pltpu.sync_copy(idx_hbm.at[pl.ds(base, rows_per_w)], idx_v)

        def shift_body(i, carry):
            v = idx_v[pl.ds(i * 16, 16)]
            pidx_v[pl.ds(i * 16, 16)] = lax.shift_right_logical(v, 1)
            return carry

        lax.fori_loop(0, rows_per_w // 16, shift_body, 0)

        iota = lax.iota(jnp.int32, 16)
        rowids = [iota + 16 * jg for jg in range(8)]

        def issue(t, b):
            pltpu.async_copy(
                pairs_hbm.at[pidx_v.at[pl.ds(t * CHUNK, CHUNK)]], bufs[b], sems[b]
            )

        for b in range(NBUF):
            issue(b, b)

        def process(t, b, tb):
            pltpu.make_async_copy(
                pairs_hbm.at[pidx_v.at[pl.ds(t * CHUNK, CHUNK)]], bufs[b], sems[b]
            ).wait()
            colbs = []
            for jg in range(8):
                iv = idx_v[pl.ds(t * CHUNK + jg * 16, 16)]
                colbs.append((iv & 1) * HIDDEN)

            def hbody(h4, carry):
                for dh in range(4):
                    h = h4 * 4 + dh
                    for jg in range(8):
                        v = plsc.load_gather(bufs[b], [rowids[jg], colbs[jg] + h])
                        tbufs[tb][h, pl.ds(jg * 16, 16)] = v
                return carry

            lax.fori_loop(0, HIDDEN // 4, hbody, 0)
            gc = wid * ng + t
            ds = gc // npg
            b0 = (gc % npg) * CHUNK
            pltpu.sync_copy(tbufs[tb], out_hbm.at[ds, :, pl.ds(b0, CHUNK)])

        def outer(i, carry):
            for k in range(NBUF):
                t = i * NBUF + k
                process(t, k, k % 2)

                @pl.when(t + NBUF < ng)
                def _():
                    issue(t + NBUF, k)

            return carry

        lax.fori_loop(0, ng // NBUF, outer, 0)

    return gather_kernel


def kernel(input_var, emb_weight):
    b, d, s = input_var.shape
    v, hid = emb_weight.shape
    idxt = jnp.transpose(input_var, (1, 2, 0)).reshape(-1).astype(jnp.int32)
    pairs = emb_weight.reshape(v // 2, 2 * hid)
    out3 = _make_gather(idxt.shape[0], d * s, b)(idxt, pairs)
    out4 = out3.reshape(d, s, hid, b)
    return jnp.transpose(out4, (3, 0, 1, 2))


# speedup vs baseline: 1.3024x; 1.3024x over previous
import functools

import jax
import jax.numpy as jnp
from jax import lax
from jax.experimental import pallas as pl
from jax.experimental.pallas import tpu as pltpu
from jax.experimental.pallas import tpu_sc as plsc

HIDDEN = 64
CHUNK = 128
NBUF = 4


@functools.lru_cache(maxsize=None)
def _make_gather(B, DS, NB):
    info = plsc.get_sparse_core_info()
    nc, ns = info.num_cores, info.num_subcores
    nw = nc * ns
    rows_per_w = B // nw
    ng = rows_per_w // CHUNK
    npg = NB // CHUNK
    assert rows_per_w % CHUNK == 0 and ng % NBUF == 0 and NB % CHUNK == 0

    mesh = plsc.VectorSubcoreMesh(core_axis_name="c", subcore_axis_name="s")

    @functools.partial(
        pl.kernel,
        mesh=mesh,
        compiler_params=pltpu.CompilerParams(needs_layout_passes=False),
        out_type=jax.ShapeDtypeStruct((DS, HIDDEN, NB), jnp.float32),
        scratch_types=(
            [pltpu.VMEM((rows_per_w,), jnp.int32)]
            + [pltpu.VMEM((rows_per_w,), jnp.int32)]
            + [pltpu.VMEM((CHUNK, 2 * HIDDEN), jnp.float32) for _ in range(NBUF)]
            + [pltpu.VMEM((HIDDEN, CHUNK), jnp.float32) for _ in range(2)]
            + [pltpu.SemaphoreType.DMA for _ in range(NBUF)]
        ),
    )
    def gather_kernel(idx_hbm, pairs_hbm, out_hbm, idx_v, pidx_v, *rest):
        bufs = rest[:NBUF]
        tbufs = rest[NBUF:NBUF + 2]
        sems = rest[NBUF + 2:]
        wid = lax.axis_index("s") * nc + lax.axis_index("c")
        base = wid * rows_per_w

        pltpu.sync_copy(idx_hbm.at[pl.ds(base, rows_per_w)], idx_v)

        @plsc.parallel_loop(0, rows_per_w // 16, unroll=8)
        def _(i):
            v = idx_v[pl.ds(i * 16, 16)]
            pidx_v[pl.ds(i * 16, 16)] = lax.shift_right_logical(v, 1)

        iota = lax.iota(jnp.int32, 16)
        rowids = [iota + 16 * jg for jg in range(8)]

        def issue(t, b):
            pltpu.async_copy(
                pairs_hbm.at[pidx_v.at[pl.ds(t * CHUNK, CHUNK)]], bufs[b], sems[b]
            )

        for b in range(NBUF):
            issue(b, b)

        def process(t, b, tb):
            pltpu.make_async_copy(
                pairs_hbm.at[pidx_v.at[pl.ds(t * CHUNK, CHUNK)]], bufs[b], sems[b]
            ).wait()
            colbs = []
            for jg in range(8):
                iv = idx_v[pl.ds(t * CHUNK + jg * 16, 16)]
                colbs.append((iv & 1) * HIDDEN)

            @plsc.parallel_loop(0, HIDDEN, unroll=8)
            def _(h):
                for jg in range(8):
                    v = plsc.load_gather(bufs[b], [rowids[jg], colbs[jg] + h])
                    tbufs[tb][h, pl.ds(jg * 16, 16)] = v
            gc = wid * ng + t
            ds = gc // npg
            b0 = (gc % npg) * CHUNK
            pltpu.sync_copy(tbufs[tb], out_hbm.at[ds, :, pl.ds(b0, CHUNK)])

        def outer(i, carry):
            for k in range(NBUF):
                t = i * NBUF + k
                process(t, k, k % 2)

                @pl.when(t + NBUF < ng)
                def _():
                    issue(t + NBUF, k)

            return carry

        lax.fori_loop(0, ng // NBUF, outer, 0)

    return gather_kernel


def kernel(input_var, emb_weight):
    b, d, s = input_var.shape
    v, hid = emb_weight.shape
    idxt = jnp.transpose(input_var, (1, 2, 0)).reshape(-1).astype(jnp.int32)
    pairs = emb_weight.reshape(v // 2, 2 * hid)
    out3 = _make_gather(idxt.shape[0], d * s, b)(idxt, pairs)
    out4 = out3.reshape(d, s, hid, b)
    return jnp.transpose(out4, (3, 0, 1, 2))


# speedup vs baseline: 1.3034x; 1.0008x over previous
import functools

import jax
import jax.numpy as jnp
from jax import lax
from jax.experimental import pallas as pl
from jax.experimental.pallas import tpu as pltpu
from jax.experimental.pallas import tpu_sc as plsc

HIDDEN = 64
CHUNK = 128
NBUF = 4


@functools.lru_cache(maxsize=None)
def _make_gather(B, DS, NB):
    info = plsc.get_sparse_core_info()
    nc, ns = info.num_cores, info.num_subcores
    nw = nc * ns
    rows_per_w = B // nw
    ng = rows_per_w // CHUNK
    npg = NB // CHUNK
    assert rows_per_w % CHUNK == 0 and ng % NBUF == 0 and NB % CHUNK == 0

    mesh = plsc.VectorSubcoreMesh(core_axis_name="c", subcore_axis_name="s")

    @functools.partial(
        pl.kernel,
        mesh=mesh,
        compiler_params=pltpu.CompilerParams(needs_layout_passes=False),
        out_type=jax.ShapeDtypeStruct((DS, HIDDEN, NB), jnp.float32),
        scratch_types=(
            [pltpu.VMEM((rows_per_w,), jnp.int32)]
            + [pltpu.VMEM((rows_per_w,), jnp.int32)]
            + [pltpu.VMEM((CHUNK, 2 * HIDDEN), jnp.float32) for _ in range(NBUF)]
            + [pltpu.VMEM((HIDDEN, CHUNK), jnp.float32) for _ in range(2)]
            + [pltpu.SemaphoreType.DMA for _ in range(NBUF)]
        ),
    )
    def gather_kernel(idx_hbm, pairs_hbm, out_hbm, idx_v, pidx_v, *rest):
        bufs = rest[:NBUF]
        tbufs = rest[NBUF:NBUF + 2]
        sems = rest[NBUF + 2:]
        wid = lax.axis_index("s") * nc + lax.axis_index("c")
        base = wid * rows_per_w

        pltpu.sync_copy(idx_hbm.at[pl.ds(base, rows_per_w)], idx_v)

        @plsc.parallel_loop(0, rows_per_w // 16, unroll=8)
        def _(i):
            v = idx_v[pl.ds(i * 16, 16)]
            pidx_v[pl.ds(i * 16, 16)] = lax.shift_right_logical(v, 1)

        iota = lax.iota(jnp.int32, 16)
        rowids = [iota + 16 * jg for jg in range(8)]

        def issue(t, b):
            pltpu.async_copy(
                pairs_hbm.at[pidx_v.at[pl.ds(t * CHUNK, CHUNK)]], bufs[b], sems[b]
            )

        for b in range(NBUF):
            issue(b, b)

        def process(t, b, tb):
            pltpu.make_async_copy(
                pairs_hbm.at[pidx_v.at[pl.ds(t * CHUNK, CHUNK)]], bufs[b], sems[b]
            ).wait()
            colbs = []
            for jg in range(8):
                iv = idx_v[pl.ds(t * CHUNK + jg * 16, 16)]
                colbs.append((iv & 1) * HIDDEN)

            @plsc.parallel_loop(0, HIDDEN, unroll=16)
            def _(h):
                for jg in range(8):
                    v = plsc.load_gather(bufs[b], [rowids[jg], colbs[jg] + h])
                    tbufs[tb][h, pl.ds(jg * 16, 16)] = v
            gc = wid * ng + t
            ds = gc // npg
            b0 = (gc % npg) * CHUNK
            pltpu.sync_copy(tbufs[tb], out_hbm.at[ds, :, pl.ds(b0, CHUNK)])

        def outer(i, carry):
            for k in range(NBUF):
                t = i * NBUF + k
                process(t, k, k % 2)

                @pl.when(t + NBUF < ng)
                def _():
                    issue(t + NBUF, k)

            return carry

        lax.fori_loop(0, ng // NBUF, outer, 0)

    return gather_kernel


def kernel(input_var, emb_weight):
    b, d, s = input_var.shape
    v, hid = emb_weight.shape
    idxt = jnp.transpose(input_var, (1, 2, 0)).reshape(-1).astype(jnp.int32)
    pairs = emb_weight.reshape(v // 2, 2 * hid)
    out3 = _make_gather(idxt.shape[0], d * s, b)(idxt, pairs)
    out4 = out3.reshape(d, s, hid, b)
    return jnp.transpose(out4, (3, 0, 1, 2))


# speedup vs baseline: 1.3510x; 1.0365x over previous
import functools

import jax
import jax.numpy as jnp
from jax import lax
from jax.experimental import pallas as pl
from jax.experimental.pallas import tpu as pltpu
from jax.experimental.pallas import tpu_sc as plsc

HIDDEN = 64
CHUNK = 128
NBUF = 4


@functools.lru_cache(maxsize=None)
def _make_gather(B, DS, NB):
    info = plsc.get_sparse_core_info()
    nc, ns = info.num_cores, info.num_subcores
    nw = nc * ns
    rows_per_w = B // nw
    ng = rows_per_w // CHUNK
    npg = NB // CHUNK
    assert rows_per_w % CHUNK == 0 and ng % NBUF == 0 and NB % CHUNK == 0

    mesh = plsc.VectorSubcoreMesh(core_axis_name="c", subcore_axis_name="s")

    @functools.partial(
        pl.kernel,
        mesh=mesh,
        compiler_params=pltpu.CompilerParams(needs_layout_passes=False),
        out_type=jax.ShapeDtypeStruct((DS, HIDDEN, NB), jnp.float32),
        scratch_types=(
            [pltpu.VMEM((rows_per_w,), jnp.int32)]
            + [pltpu.VMEM((rows_per_w,), jnp.int32)]
            + [pltpu.VMEM((CHUNK, 2 * HIDDEN), jnp.float32) for _ in range(NBUF)]
            + [pltpu.VMEM((HIDDEN, CHUNK), jnp.float32) for _ in range(2)]
            + [pltpu.SemaphoreType.DMA for _ in range(NBUF)]
            + [pltpu.SemaphoreType.DMA for _ in range(2)]
        ),
    )
    def gather_kernel(idx_hbm, pairs_hbm, out_hbm, idx_v, pidx_v, *rest):
        bufs = rest[:NBUF]
        tbufs = rest[NBUF:NBUF + 2]
        sems = rest[NBUF + 2:NBUF + 2 + NBUF]
        wsems = rest[NBUF + 2 + NBUF:]
        wid = lax.axis_index("s") * nc + lax.axis_index("c")
        base = wid * rows_per_w

        pltpu.sync_copy(idx_hbm.at[pl.ds(base, rows_per_w)], idx_v)

        @plsc.parallel_loop(0, rows_per_w // 16, unroll=8)
        def _(i):
            v = idx_v[pl.ds(i * 16, 16)]
            pidx_v[pl.ds(i * 16, 16)] = lax.shift_right_logical(v, 1)

        iota = lax.iota(jnp.int32, 16)
        rowids = [iota + 16 * jg for jg in range(8)]

        def issue(t, b):
            pltpu.async_copy(
                pairs_hbm.at[pidx_v.at[pl.ds(t * CHUNK, CHUNK)]], bufs[b], sems[b]
            )

        for b in range(NBUF):
            issue(b, b)

        def out_slice(t):
            gc = wid * ng + t
            ds = gc // npg
            b0 = (gc % npg) * CHUNK
            return out_hbm.at[ds, :, pl.ds(b0, CHUNK)]

        def process(t, b, tb):
            pltpu.make_async_copy(
                pairs_hbm.at[pidx_v.at[pl.ds(t * CHUNK, CHUNK)]], bufs[b], sems[b]
            ).wait()

            @pl.when(t >= 2)
            def _():
                pltpu.make_async_copy(tbufs[tb], out_slice(t), wsems[tb]).wait()

            colbs = []
            for jg in range(8):
                iv = idx_v[pl.ds(t * CHUNK + jg * 16, 16)]
                colbs.append((iv & 1) * HIDDEN)

            @plsc.parallel_loop(0, HIDDEN, unroll=16)
            def _(h):
                for jg in range(8):
                    v = plsc.load_gather(bufs[b], [rowids[jg], colbs[jg] + h])
                    tbufs[tb][h, pl.ds(jg * 16, 16)] = v

            pltpu.async_copy(tbufs[tb], out_slice(t), wsems[tb])

        def outer(i, carry):
            for k in range(NBUF):
                t = i * NBUF + k
                process(t, k, k % 2)

                @pl.when(t + NBUF < ng)
                def _():
                    issue(t + NBUF, k)

            return carry

        lax.fori_loop(0, ng // NBUF, outer, 0)

        for tb in range(2):
            pltpu.make_async_copy(
                tbufs[tb], out_slice(ng - 2 + tb), wsems[tb]
            ).wait()

    return gather_kernel


def kernel(input_var, emb_weight):
    b, d, s = input_var.shape
    v, hid = emb_weight.shape
    idxt = jnp.transpose(input_var, (1, 2, 0)).reshape(-1).astype(jnp.int32)
    pairs = emb_weight.reshape(v // 2, 2 * hid)
    out3 = _make_gather(idxt.shape[0], d * s, b)(idxt, pairs)
    out4 = out3.reshape(d, s, hid, b)
    return jnp.transpose(out4, (3, 0, 1, 2))
